# baseline (speedup 1.0000x reference)
import jax
import jax.numpy as jnp
from jax import lax
from jax.experimental import pallas as pl
from jax.experimental.pallas import tpu as pltpu

DIM = 128
EMB = 3 * DIM
IN_DIM = 387
HID = 512
OUT = 128
Q = HID // 4


def _body(c_hbm, s_hbm, i_hbm, s0_hbm, s1_hbm, s2_hbm,
          cat_hbm, sub_hbm, ind_hbm, w1_hbm, w2_hbm, b1_hbm, b2_hbm,
          out_r, w1_v, w2_v, b1_v, b2_v, emb_v,
          c_sm, s_sm, i_sm, s0_sm, s1_sm, s2_sm,
          semw1, semw2, semb, semi, semx):
    cw1 = [pltpu.make_async_copy(w1_hbm.at[pl.ds(q * Q, Q), :],
                                 w1_v.at[pl.ds(q * Q, Q), :], semw1)
           for q in range(4)]
    cw2 = pltpu.make_async_copy(w2_hbm, w2_v, semw2)
    cb1 = pltpu.make_async_copy(b1_hbm, b1_v, semb)
    cb2 = pltpu.make_async_copy(b2_hbm, b2_v, semb)
    ci = [pltpu.make_async_copy(src, dst, semi)
          for src, dst in ((c_hbm, c_sm), (s_hbm, s_sm), (i_hbm, i_sm),
                           (s0_hbm, s0_sm), (s1_hbm, s1_sm), (s2_hbm, s2_sm))]
    for c in cw1:
        c.start()
    cw2.start()
    cb1.start()
    cb2.start()
    for c in ci:
        c.start()
    for c in ci:
        c.wait()

    cx0 = pltpu.make_async_copy(cat_hbm.at[pl.ds(c_sm[0], 1), :],
                                emb_v.at[:, pl.ds(0, DIM)], semx)
    cx1 = pltpu.make_async_copy(sub_hbm.at[pl.ds(s_sm[0], 1), :],
                                emb_v.at[:, pl.ds(DIM, DIM)], semx)
    cx2 = pltpu.make_async_copy(ind_hbm.at[pl.ds(i_sm[0], 1), :],
                                emb_v.at[:, pl.ds(2 * DIM, DIM)], semx)
    cx0.start()
    cx1.start()
    cx2.start()

    lane = lax.broadcasted_iota(jnp.int32, (1, DIM), 1)
    tail = jnp.where(lane == 0, s0_sm[0],
           jnp.where(lane == 1, s1_sm[0],
           jnp.where(lane == 2, s2_sm[0], 0.0)))
    emb_v[:, pl.ds(EMB, DIM)] = tail

    cx0.wait()
    cx1.wait()
    cx2.wait()
    for c in cw1:
        c.wait()
    cb1.wait()
    h = lax.dot_general(emb_v[:, pl.ds(0, IN_DIM)], w1_v[...],
                        (((1,), (1,)), ((), ())),
                        preferred_element_type=jnp.float32)
    h = jnp.maximum(h + b1_v[...][None, :], 0.0)

    cw2.wait()
    cb2.wait()
    out = lax.dot_general(h, w2_v[...], (((1,), (1,)), ((), ())),
                          preferred_element_type=jnp.float32)
    out_r[...] = out[0] + b2_v[...]


@jax.jit
def _run(c_i, s_i, i_i, s0, s1, s2,
         cat_table, sub_table, ind_table, W1, W2, b1, b2):
    f = pl.pallas_call(
        _body,
        in_specs=[pl.BlockSpec(memory_space=pltpu.HBM)] * 13,
        out_shape=jax.ShapeDtypeStruct((OUT,), jnp.float32),
        scratch_shapes=[
            pltpu.VMEM((HID, IN_DIM), jnp.float32),
            pltpu.VMEM((OUT, HID), jnp.float32),
            pltpu.VMEM((HID,), jnp.float32),
            pltpu.VMEM((OUT,), jnp.float32),
            pltpu.VMEM((1, HID), jnp.float32),
            pltpu.SMEM((1,), jnp.int32),
            pltpu.SMEM((1,), jnp.int32),
            pltpu.SMEM((1,), jnp.int32),
            pltpu.SMEM((1,), jnp.float32),
            pltpu.SMEM((1,), jnp.float32),
            pltpu.SMEM((1,), jnp.float32),
            pltpu.SemaphoreType.DMA,
            pltpu.SemaphoreType.DMA,
            pltpu.SemaphoreType.DMA,
            pltpu.SemaphoreType.DMA,
            pltpu.SemaphoreType.DMA,
        ],
        name="project_encoder_tc",
    )
    big = [cat_table, sub_table, ind_table, W1, W2, b1, b2]
    big = [pltpu.with_memory_space_constraint(a, pltpu.HBM) for a in big]
    return f(c_i, s_i, i_i, s0, s1, s2, *big)


def kernel(category, sub_category, industry, average_score, client_feedback,
           total_awards_and_tips, cat_table, sub_table, ind_table,
           W1, b1, W2, b2):
    return _run(category[None], sub_category[None], industry[None],
                average_score, client_feedback, total_awards_and_tips,
                cat_table, sub_table, ind_table, W1, W2, b1, b2)


# speedup vs baseline: 1.4395x; 1.4395x over previous
import jax
import jax.numpy as jnp
from jax import lax
from jax.experimental import pallas as pl
from jax.experimental.pallas import tpu as pltpu

DIM = 128
EMB = 3 * DIM
IN_DIM = 387
HID = 512
OUT = 128
Q = HID // 4


def _body(c_hbm, s_hbm, i_hbm, s0_hbm, s1_hbm, s2_hbm,
          cat_hbm, sub_hbm, ind_hbm, w1_hbm, w2_hbm, b1_hbm, b2_hbm,
          out_r, w1_v, w2_v, b1_v, b2_v, emb_v,
          c_sm, s_sm, i_sm, s0_sm, s1_sm, s2_sm,
          semw1, semw2, semb, semi, semx):
    cw1 = [pltpu.make_async_copy(w1_hbm.at[pl.ds(0, 192), :],
                                 w1_v.at[pl.ds(0, 192), :], semw1),
           pltpu.make_async_copy(w1_hbm.at[pl.ds(192, IN_DIM - 192), :],
                                 w1_v.at[pl.ds(192, IN_DIM - 192), :], semw1)]
    cw2 = pltpu.make_async_copy(w2_hbm, w2_v, semw2)
    cb1 = pltpu.make_async_copy(b1_hbm, b1_v, semb)
    cb2 = pltpu.make_async_copy(b2_hbm, b2_v, semb)
    ci = [pltpu.make_async_copy(src, dst, semi)
          for src, dst in ((c_hbm, c_sm), (s_hbm, s_sm), (i_hbm, i_sm),
                           (s0_hbm, s0_sm), (s1_hbm, s1_sm), (s2_hbm, s2_sm))]
    for c in cw1:
        c.start()
    cw2.start()
    cb1.start()
    cb2.start()
    for c in ci:
        c.start()
    for c in ci:
        c.wait()

    cx0 = pltpu.make_async_copy(cat_hbm.at[pl.ds(c_sm[0], 1), :],
                                emb_v.at[:, pl.ds(0, DIM)], semx)
    cx1 = pltpu.make_async_copy(sub_hbm.at[pl.ds(s_sm[0], 1), :],
                                emb_v.at[:, pl.ds(DIM, DIM)], semx)
    cx2 = pltpu.make_async_copy(ind_hbm.at[pl.ds(i_sm[0], 1), :],
                                emb_v.at[:, pl.ds(2 * DIM, DIM)], semx)
    cx0.start()
    cx1.start()
    cx2.start()

    lane = lax.broadcasted_iota(jnp.int32, (1, DIM), 1)
    tail = jnp.where(lane == 0, s0_sm[0],
           jnp.where(lane == 1, s1_sm[0],
           jnp.where(lane == 2, s2_sm[0], 0.0)))
    emb_v[:, pl.ds(EMB, DIM)] = tail

    cx0.wait()
    cx1.wait()
    cx2.wait()
    for c in cw1:
        c.wait()
    cb1.wait()
    h = lax.dot_general(emb_v[:, pl.ds(0, IN_DIM)], w1_v[...],
                        (((1,), (0,)), ((), ())),
                        preferred_element_type=jnp.float32)
    h = jnp.maximum(h + b1_v[...][None, :], 0.0)

    cw2.wait()
    cb2.wait()
    out = lax.dot_general(h, w2_v[...], (((1,), (1,)), ((), ())),
                          preferred_element_type=jnp.float32)
    out_r[...] = out[0] + b2_v[...]


@jax.jit
def _run(c_i, s_i, i_i, s0, s1, s2,
         cat_table, sub_table, ind_table, W1, W2, b1, b2):
    f = pl.pallas_call(
        _body,
        in_specs=[pl.BlockSpec(memory_space=pltpu.HBM)] * 13,
        out_shape=jax.ShapeDtypeStruct((OUT,), jnp.float32),
        scratch_shapes=[
            pltpu.VMEM((IN_DIM, HID), jnp.float32),
            pltpu.VMEM((OUT, HID), jnp.float32),
            pltpu.VMEM((HID,), jnp.float32),
            pltpu.VMEM((OUT,), jnp.float32),
            pltpu.VMEM((1, HID), jnp.float32),
            pltpu.SMEM((1,), jnp.int32),
            pltpu.SMEM((1,), jnp.int32),
            pltpu.SMEM((1,), jnp.int32),
            pltpu.SMEM((1,), jnp.float32),
            pltpu.SMEM((1,), jnp.float32),
            pltpu.SMEM((1,), jnp.float32),
            pltpu.SemaphoreType.DMA,
            pltpu.SemaphoreType.DMA,
            pltpu.SemaphoreType.DMA,
            pltpu.SemaphoreType.DMA,
            pltpu.SemaphoreType.DMA,
        ],
        name="project_encoder_tc",
    )
    big = [cat_table, sub_table, ind_table, W1.T, W2, b1, b2]
    big = [pltpu.with_memory_space_constraint(a, pltpu.HBM) for a in big]
    return f(c_i, s_i, i_i, s0, s1, s2, *big)


def kernel(category, sub_category, industry, average_score, client_feedback,
           total_awards_and_tips, cat_table, sub_table, ind_table,
           W1, b1, W2, b2):
    return _run(category[None], sub_category[None], industry[None],
                average_score, client_feedback, total_awards_and_tips,
                cat_table, sub_table, ind_table, W1, W2, b1, b2)
